# baseline (speedup 1.0000x reference)
import functools
from itertools import combinations

import numpy as np
import jax
import jax.numpy as jnp
from jax import lax
from jax.experimental import pallas as pl
from jax.experimental.pallas import tpu as pltpu
from jax.experimental.pallas import tpu_sc as plsc

KC = 20
NBLK = 5
NGROUP = 112
NLANE = 16
NHIST = NGROUP * NLANE
NWORKER = 32
BT = 65536
CHK = 8192
NSUB = 240
INV_LOG2 = 1.4426950408889634


def _build_maps():
    mt = np.zeros((NGROUP, NSUB), np.float32)
    subs4 = list(combinations(range(4), 3))
    for i in range(NBLK):
        for nib in range(16):
            bits = [(nib >> m) & 1 for m in range(4)]
            for s, sub in enumerate(subs4):
                key = bits[sub[0]] + 2 * bits[sub[1]] + 4 * bits[sub[2]]
                mt[i * 16 + nib, i * 32 + s * 8 + key] += 1.0
    subs5 = list(combinations(range(5), 3))
    for g in range(32):
        bits = [(g >> m) & 1 for m in range(5)]
        for s, sub in enumerate(subs5):
            key = bits[sub[0]] + 2 * bits[sub[1]] + 4 * bits[sub[2]]
            mt[80 + g, 160 + s * 8 + key] += 1.0
    gt = np.zeros((NSUB, 8), np.float32)
    for i in range(NBLK):
        gt[i * 32:(i + 1) * 32, i] = 0.25
    gt[160:NSUB, 5] = 0.1
    return mt, gt


_MT, _GT = _build_maps()


@functools.lru_cache(maxsize=None)
def _tc_pack_fn(nrows):
    def body(x_ref, out_ref):
        x = x_ref[...]
        jj = lax.broadcasted_iota(jnp.int32, (KC, 1), 0)
        wcol = (jnp.int32(1) << jj).astype(jnp.float32)
        t = jnp.where(x > 0, wcol, 0.0)
        acc = jnp.sum(t, axis=0).astype(jnp.int32)
        for i in range(NBLK):
            s = jnp.sum(x[4 * i:4 * i + 4, :], axis=0)
            acc = acc + jnp.where(s > 0, np.int32(1 << (20 + i)), np.int32(0))
        out_ref[...] = acc

    return pl.pallas_call(
        body,
        grid=(nrows // BT,),
        in_specs=[pl.BlockSpec((KC, BT), lambda i: (0, i))],
        out_specs=pl.BlockSpec((BT,), lambda i: (i,)),
        out_shape=jax.ShapeDtypeStruct((nrows,), jnp.int32),
    )


@functools.lru_cache(maxsize=None)
def _sc_hist_fn(nrows):
    rows_w = nrows // NWORKER
    nchunk = rows_w // CHK
    mesh = plsc.VectorSubcoreMesh(core_axis_name="c", subcore_axis_name="s")

    @functools.partial(
        pl.kernel,
        mesh=mesh,
        compiler_params=pltpu.CompilerParams(
            needs_layout_passes=False, use_tc_tiling_on_sc=False),
        out_type=jax.ShapeDtypeStruct((NWORKER * NHIST,), jnp.float32),
        scratch_types=[
            pltpu.VMEM((CHK,), jnp.int32),
            pltpu.VMEM((CHK,), jnp.int32),
            pltpu.VMEM((NHIST,), jnp.float32),
            pltpu.SemaphoreType.DMA,
            pltpu.SemaphoreType.DMA,
        ],
    )
    def sc_hist(pk_hbm, out_hbm, buf0, buf1, hist, sem0, sem1):
        wid = lax.axis_index("s") * 2 + lax.axis_index("c")
        base = wid * rows_w
        zeros16 = jnp.zeros((NLANE,), jnp.float32)
        for b in range(NGROUP):
            hist[pl.ds(b * NLANE, NLANE)] = zeros16
        lane = lax.iota(jnp.int32, NLANE)
        ones16 = jnp.ones((NLANE,), jnp.float32)

        def compute(buf):
            def body(t, carry):
                p = buf[pl.ds(t * NLANE, NLANE)]
                plsc.addupdate_scatter(
                    hist, [((p & 15) << 4) + lane], ones16)
                for i in range(1, NBLK):
                    plsc.addupdate_scatter(
                        hist,
                        [((p >> (4 * i - 4)) & 240) + (lane + i * 256)],
                        ones16)
                plsc.addupdate_scatter(
                    hist, [((p >> 16) & 496) + (lane + 1280)], ones16)
                return carry

            lax.fori_loop(0, CHK // NLANE, body, 0)

        bufs = (buf0, buf1)
        sems = (sem0, sem1)
        handles = [None] * nchunk
        handles[0] = pltpu.async_copy(
            pk_hbm.at[pl.ds(base, CHK)], bufs[0], sems[0])
        for ci in range(nchunk):
            if ci + 1 < nchunk:
                nb = (ci + 1) & 1
                handles[ci + 1] = pltpu.async_copy(
                    pk_hbm.at[pl.ds(base + (ci + 1) * CHK, CHK)],
                    bufs[nb], sems[nb])
            handles[ci].wait()
            compute(bufs[ci & 1])
        pltpu.sync_copy(hist, out_hbm.at[pl.ds(wid * NHIST, NHIST)])

    return sc_hist


@functools.lru_cache(maxsize=None)
def _tc_entropy_fn(nrows):
    denom = np.float32(nrows + 1e-8)

    def body(p_ref, mt_ref, gt_ref, out_ref):
        x = p_ref[...]
        c = jnp.sum(jnp.sum(x, axis=2), axis=0)
        counts = c.reshape(1, NGROUP)
        sub = jnp.dot(counts, mt_ref[...], preferred_element_type=jnp.float32,
                      precision=jax.lax.Precision.HIGHEST)
        p = sub / denom
        logp = jnp.log(jnp.where(sub > 0, p, 1.0)) * INV_LOG2
        t = jnp.where(sub > 0, -(p * logp), 0.0)
        out_ref[...] = jnp.dot(t, gt_ref[...], preferred_element_type=jnp.float32,
                               precision=jax.lax.Precision.HIGHEST)

    return pl.pallas_call(
        body,
        out_shape=jax.ShapeDtypeStruct((1, 8), jnp.float32),
    )


def kernel(phi):
    nrows = phi.shape[0]
    packed = _tc_pack_fn(nrows)(phi.T)
    partial = _sc_hist_fn(nrows)(packed)
    partial3 = partial.reshape(NWORKER, NGROUP, NLANE)
    res = _tc_entropy_fn(nrows)(partial3, _MT, _GT)
    return res[0, :NBLK], res[0, NBLK]


# speedup vs baseline: 1.0790x; 1.0790x over previous
import functools
from itertools import combinations

import numpy as np
import jax
import jax.numpy as jnp
from jax import lax
from jax.experimental import pallas as pl
from jax.experimental.pallas import tpu as pltpu
from jax.experimental.pallas import tpu_sc as plsc

KC = 20
NBLK = 5
NGROUP = 112
NLANE = 16
NHIST = NGROUP * NLANE
NWORKER = 32
BT = 32768
CHK = 8192
NSUB = 240
INV_LOG2 = 1.4426950408889634


def _build_maps():
    mt = np.zeros((NGROUP, NSUB), np.float32)
    subs4 = list(combinations(range(4), 3))
    for i in range(NBLK):
        for nib in range(16):
            bits = [(nib >> m) & 1 for m in range(4)]
            for s, sub in enumerate(subs4):
                key = bits[sub[0]] + 2 * bits[sub[1]] + 4 * bits[sub[2]]
                mt[i * 16 + nib, i * 32 + s * 8 + key] += 1.0
    subs5 = list(combinations(range(5), 3))
    for g in range(32):
        bits = [(g >> m) & 1 for m in range(5)]
        for s, sub in enumerate(subs5):
            key = bits[sub[0]] + 2 * bits[sub[1]] + 4 * bits[sub[2]]
            mt[80 + g, 160 + s * 8 + key] += 1.0
    gt = np.zeros((NSUB, 8), np.float32)
    for i in range(NBLK):
        gt[i * 32:(i + 1) * 32, i] = 0.25
    gt[160:NSUB, 5] = 0.1
    return mt, gt


_MT, _GT = _build_maps()


@functools.lru_cache(maxsize=None)
def _tc_pack_fn(nrows):
    def body(x_ref, out_ref):
        jj8 = lax.broadcasted_iota(jnp.int32, (8, 1), 0)
        acc_f = None
        for k in range(2):
            xk = x_ref[8 * k:8 * k + 8, :]
            wcol = (jnp.int32(1) << (jj8 + 8 * k)).astype(jnp.float32)
            tk = jnp.where(xk > 0, wcol, 0.0)
            y = xk + pltpu.roll(xk, 7, 0)
            z = y + pltpu.roll(y, 6, 0)
            w5 = (jnp.where(jj8 == 0, float(1 << (20 + 2 * k)), 0.0)
                  + jnp.where(jj8 == 4, float(1 << (21 + 2 * k)), 0.0))
            t5 = jnp.where(z > 0, w5, 0.0)
            tk = tk + t5
            acc_f = tk if acc_f is None else acc_f + tk
        x2 = x_ref[16:KC, :]
        jj4 = lax.broadcasted_iota(jnp.int32, (4, 1), 0)
        wcol2 = (jnp.int32(1) << (jj4 + 16)).astype(jnp.float32)
        t2 = jnp.where(x2 > 0, wcol2, 0.0)
        y2 = x2 + pltpu.roll(x2, 3, 0)
        z2 = y2 + pltpu.roll(y2, 2, 0)
        w52 = jnp.where(jj4 == 0, float(1 << 24), 0.0)
        t52 = jnp.where(z2 > 0, w52, 0.0)
        lo = jnp.sum(acc_f, axis=0) + jnp.sum(t2, axis=0)
        hi = jnp.sum(t52, axis=0)
        out_ref[...] = lo.astype(jnp.int32) + hi.astype(jnp.int32)

    return pl.pallas_call(
        body,
        grid=(nrows // BT,),
        in_specs=[pl.BlockSpec((KC, BT), lambda i: (0, i))],
        out_specs=pl.BlockSpec((BT,), lambda i: (i,)),
        out_shape=jax.ShapeDtypeStruct((nrows,), jnp.int32),
    )


@functools.lru_cache(maxsize=None)
def _sc_hist_fn(nrows):
    rows_w = nrows // NWORKER
    nchunk = rows_w // CHK
    mesh = plsc.VectorSubcoreMesh(core_axis_name="c", subcore_axis_name="s")

    @functools.partial(
        pl.kernel,
        mesh=mesh,
        compiler_params=pltpu.CompilerParams(
            needs_layout_passes=False, use_tc_tiling_on_sc=False),
        out_type=jax.ShapeDtypeStruct((NWORKER * NHIST,), jnp.float32),
        scratch_types=[
            pltpu.VMEM((CHK,), jnp.int32),
            pltpu.VMEM((CHK,), jnp.int32),
            pltpu.VMEM((NHIST,), jnp.float32),
            pltpu.SemaphoreType.DMA,
            pltpu.SemaphoreType.DMA,
        ],
    )
    def sc_hist(pk_hbm, out_hbm, buf0, buf1, hist, sem0, sem1):
        wid = lax.axis_index("s") * 2 + lax.axis_index("c")
        base = wid * rows_w
        zeros16 = jnp.zeros((NLANE,), jnp.float32)
        for b in range(NGROUP):
            hist[pl.ds(b * NLANE, NLANE)] = zeros16
        lane = lax.iota(jnp.int32, NLANE)
        ones16 = jnp.ones((NLANE,), jnp.float32)

        def compute(buf):
            def body(t, carry):
                p = buf[pl.ds(t * NLANE, NLANE)]
                plsc.addupdate_scatter(
                    hist, [((p & 15) << 4) + lane], ones16)
                for i in range(1, NBLK):
                    plsc.addupdate_scatter(
                        hist,
                        [((p >> (4 * i - 4)) & 240) + (lane + i * 256)],
                        ones16)
                plsc.addupdate_scatter(
                    hist, [((p >> 16) & 496) + (lane + 1280)], ones16)
                return carry

            lax.fori_loop(0, CHK // NLANE, body, 0)

        bufs = (buf0, buf1)
        sems = (sem0, sem1)
        handles = [None] * nchunk
        handles[0] = pltpu.async_copy(
            pk_hbm.at[pl.ds(base, CHK)], bufs[0], sems[0])
        for ci in range(nchunk):
            if ci + 1 < nchunk:
                nb = (ci + 1) & 1
                handles[ci + 1] = pltpu.async_copy(
                    pk_hbm.at[pl.ds(base + (ci + 1) * CHK, CHK)],
                    bufs[nb], sems[nb])
            handles[ci].wait()
            compute(bufs[ci & 1])
        pltpu.sync_copy(hist, out_hbm.at[pl.ds(wid * NHIST, NHIST)])

    return sc_hist


@functools.lru_cache(maxsize=None)
def _tc_entropy_fn(nrows):
    denom = np.float32(nrows + 1e-8)

    def body(p_ref, mt_ref, gt_ref, out_ref):
        x = p_ref[...]
        c = jnp.sum(jnp.sum(x, axis=2), axis=0)
        counts = c.reshape(1, NGROUP)
        sub = jnp.dot(counts, mt_ref[...], preferred_element_type=jnp.float32,
                      precision=jax.lax.Precision.HIGHEST)
        p = sub / denom
        logp = jnp.log(jnp.where(sub > 0, p, 1.0)) * INV_LOG2
        t = jnp.where(sub > 0, -(p * logp), 0.0)
        out_ref[...] = jnp.dot(t, gt_ref[...], preferred_element_type=jnp.float32,
                               precision=jax.lax.Precision.HIGHEST)

    return pl.pallas_call(
        body,
        out_shape=jax.ShapeDtypeStruct((1, 8), jnp.float32),
    )


def kernel(phi):
    nrows = phi.shape[0]
    packed = _tc_pack_fn(nrows)(phi.T)
    partial = _sc_hist_fn(nrows)(packed)
    partial3 = partial.reshape(NWORKER, NGROUP, NLANE)
    res = _tc_entropy_fn(nrows)(partial3, _MT, _GT)
    return res[0, :NBLK], res[0, NBLK]


# speedup vs baseline: 1.1401x; 1.0566x over previous
import functools
from itertools import combinations

import numpy as np
import jax
import jax.numpy as jnp
from jax import lax
from jax.experimental import pallas as pl
from jax.experimental.pallas import tpu as pltpu
from jax.experimental.pallas import tpu_sc as plsc

KC = 20
NBLK = 5
NGROUP = 112
NLANE = 16
NHIST = NGROUP * NLANE
NWORKER = 32
BT = 32768
CHK = 8192
NSUB = 240
INV_LOG2 = 1.4426950408889634


def _build_maps():
    mt = np.zeros((NGROUP, NSUB), np.float32)
    subs4 = list(combinations(range(4), 3))
    for i in range(NBLK):
        for nib in range(16):
            bits = [(nib >> m) & 1 for m in range(4)]
            for s, sub in enumerate(subs4):
                key = bits[sub[0]] + 2 * bits[sub[1]] + 4 * bits[sub[2]]
                mt[i * 16 + nib, i * 32 + s * 8 + key] += 1.0
    subs5 = list(combinations(range(5), 3))
    for g in range(32):
        bits = [(g >> m) & 1 for m in range(5)]
        for s, sub in enumerate(subs5):
            key = bits[sub[0]] + 2 * bits[sub[1]] + 4 * bits[sub[2]]
            mt[80 + g, 160 + s * 8 + key] += 1.0
    gt = np.zeros((NSUB, 8), np.float32)
    for i in range(NBLK):
        gt[i * 32:(i + 1) * 32, i] = 0.25
    gt[160:NSUB, 5] = 0.1
    return mt, gt


_MT, _GT = _build_maps()


@functools.lru_cache(maxsize=None)
def _tc_pack_fn(nrows):
    def body(x_ref, out_ref):
        jj8 = lax.broadcasted_iota(jnp.int32, (8, 1), 0)
        acc_f = None
        for k in range(2):
            xk = x_ref[8 * k:8 * k + 8, :]
            wcol = (jnp.int32(1) << (jj8 + 8 * k)).astype(jnp.float32)
            tk = jnp.where(xk > 0, wcol, 0.0)
            y = xk + pltpu.roll(xk, 7, 0)
            z = y + pltpu.roll(y, 6, 0)
            w5 = (jnp.where(jj8 == 0, float(1 << (20 + 2 * k)), 0.0)
                  + jnp.where(jj8 == 4, float(1 << (21 + 2 * k)), 0.0))
            t5 = jnp.where(z > 0, w5, 0.0)
            tk = tk + t5
            acc_f = tk if acc_f is None else acc_f + tk
        x2 = x_ref[16:KC, :]
        jj4 = lax.broadcasted_iota(jnp.int32, (4, 1), 0)
        wcol2 = (jnp.int32(1) << (jj4 + 16)).astype(jnp.float32)
        t2 = jnp.where(x2 > 0, wcol2, 0.0)
        y2 = x2 + pltpu.roll(x2, 3, 0)
        z2 = y2 + pltpu.roll(y2, 2, 0)
        w52 = jnp.where(jj4 == 0, float(1 << 24), 0.0)
        t52 = jnp.where(z2 > 0, w52, 0.0)
        lo = jnp.sum(acc_f, axis=0)
        hi = jnp.sum(t2 + t52, axis=0)
        out_ref[...] = lo.astype(jnp.int32) + hi.astype(jnp.int32)

    return pl.pallas_call(
        body,
        grid=(nrows // BT,),
        in_specs=[pl.BlockSpec((KC, BT), lambda i: (0, i))],
        out_specs=pl.BlockSpec((BT,), lambda i: (i,)),
        out_shape=jax.ShapeDtypeStruct((nrows,), jnp.int32),
    )


@functools.lru_cache(maxsize=None)
def _sc_hist_fn(nrows):
    rows_w = nrows // NWORKER
    nchunk = rows_w // CHK
    mesh = plsc.VectorSubcoreMesh(core_axis_name="c", subcore_axis_name="s")

    @functools.partial(
        pl.kernel,
        mesh=mesh,
        compiler_params=pltpu.CompilerParams(
            needs_layout_passes=False, use_tc_tiling_on_sc=False),
        out_type=jax.ShapeDtypeStruct((NWORKER * NHIST,), jnp.float32),
        scratch_types=[
            pltpu.VMEM((CHK,), jnp.int32),
            pltpu.VMEM((CHK,), jnp.int32),
            pltpu.VMEM((NHIST,), jnp.float32),
            pltpu.SemaphoreType.DMA,
            pltpu.SemaphoreType.DMA,
        ],
    )
    def sc_hist(pk_hbm, out_hbm, buf0, buf1, hist, sem0, sem1):
        wid = lax.axis_index("s") * 2 + lax.axis_index("c")
        base = wid * rows_w
        zeros16 = jnp.zeros((NLANE,), jnp.float32)
        for b in range(NGROUP):
            hist[pl.ds(b * NLANE, NLANE)] = zeros16
        lane = lax.iota(jnp.int32, NLANE)
        ones16 = jnp.ones((NLANE,), jnp.float32)

        def compute(buf):
            def body(t, carry):
                p = buf[pl.ds(t * NLANE, NLANE)]
                plsc.addupdate_scatter(
                    hist, [((p & 15) << 4) + lane], ones16)
                for i in range(1, NBLK):
                    plsc.addupdate_scatter(
                        hist,
                        [((p >> (4 * i - 4)) & 240) + (lane + i * 256)],
                        ones16)
                plsc.addupdate_scatter(
                    hist, [((p >> 16) & 496) + (lane + 1280)], ones16)
                return carry

            lax.fori_loop(0, CHK // NLANE, body, 0)

        bufs = (buf0, buf1)
        sems = (sem0, sem1)
        handles = [None] * nchunk
        handles[0] = pltpu.async_copy(
            pk_hbm.at[pl.ds(base, CHK)], bufs[0], sems[0])
        for ci in range(nchunk):
            if ci + 1 < nchunk:
                nb = (ci + 1) & 1
                handles[ci + 1] = pltpu.async_copy(
                    pk_hbm.at[pl.ds(base + (ci + 1) * CHK, CHK)],
                    bufs[nb], sems[nb])
            handles[ci].wait()
            compute(bufs[ci & 1])
        pltpu.sync_copy(hist, out_hbm.at[pl.ds(wid * NHIST, NHIST)])

    return sc_hist


@functools.lru_cache(maxsize=None)
def _tc_entropy_fn(nrows):
    denom = np.float32(nrows + 1e-8)

    def body(p_ref, mt_ref, gt_ref, out_ref):
        x = p_ref[...]
        c = jnp.sum(jnp.sum(x, axis=2), axis=0)
        counts = c.reshape(1, NGROUP)
        sub = jnp.dot(counts, mt_ref[...], preferred_element_type=jnp.float32,
                      precision=jax.lax.Precision.HIGHEST)
        p = sub / denom
        logp = jnp.log(jnp.where(sub > 0, p, 1.0)) * INV_LOG2
        t = jnp.where(sub > 0, -(p * logp), 0.0)
        out_ref[...] = jnp.dot(t, gt_ref[...], preferred_element_type=jnp.float32,
                               precision=jax.lax.Precision.HIGHEST)

    return pl.pallas_call(
        body,
        out_shape=jax.ShapeDtypeStruct((1, 8), jnp.float32),
    )


def kernel(phi):
    nrows = phi.shape[0]
    packed = _tc_pack_fn(nrows)(phi.T)
    partial = _sc_hist_fn(nrows)(packed)
    partial3 = partial.reshape(NWORKER, NGROUP, NLANE)
    res = _tc_entropy_fn(nrows)(partial3, _MT, _GT)
    return res[0, :NBLK], res[0, NBLK]


# speedup vs baseline: 1.1648x; 1.0216x over previous
import functools
from itertools import combinations

import numpy as np
import jax
import jax.numpy as jnp
from jax import lax
from jax.experimental import pallas as pl
from jax.experimental.pallas import tpu as pltpu
from jax.experimental.pallas import tpu_sc as plsc

KC = 20
NBLK = 5
NGROUP = 112
NLANE = 16
NHIST = NGROUP * NLANE
NWORKER = 32
BT = 32768
CHK = 8192
NSUB = 240
INV_LOG2 = 1.4426950408889634


def _build_maps():
    mt = np.zeros((NGROUP, NSUB), np.float32)
    subs4 = list(combinations(range(4), 3))
    for i in range(NBLK):
        for nib in range(16):
            bits = [(nib >> m) & 1 for m in range(4)]
            for s, sub in enumerate(subs4):
                key = bits[sub[0]] + 2 * bits[sub[1]] + 4 * bits[sub[2]]
                mt[i * 16 + nib, i * 32 + s * 8 + key] += 1.0
    subs5 = list(combinations(range(5), 3))
    for g in range(32):
        bits = [(g >> m) & 1 for m in range(5)]
        for s, sub in enumerate(subs5):
            key = bits[sub[0]] + 2 * bits[sub[1]] + 4 * bits[sub[2]]
            mt[80 + g, 160 + s * 8 + key] += 1.0
    gt = np.zeros((NSUB, 8), np.float32)
    for i in range(NBLK):
        gt[i * 32:(i + 1) * 32, i] = 0.25
    gt[160:NSUB, 5] = 0.1
    return mt, gt


_MT, _GT = _build_maps()


@functools.lru_cache(maxsize=None)
def _tc_pack_fn(nrows):
    def body(x_ref, out_ref):
        jj8 = lax.broadcasted_iota(jnp.int32, (8, 1), 0)
        zero = jnp.int32(0)
        acc = None
        for k in range(2):
            xk = x_ref[8 * k:8 * k + 8, :]
            tk = jnp.where(xk > 0, jnp.int32(1) << (jj8 + 8 * k), zero)
            y = xk + pltpu.roll(xk, 7, 0)
            z = y + pltpu.roll(y, 6, 0)
            w5 = ((jnp.where(jj8 == 0, 1, 0) << (20 + 2 * k))
                  + (jnp.where(jj8 == 4, 1, 0) << (21 + 2 * k)))
            tk = tk + jnp.where(z > 0, w5, zero)
            acc = tk if acc is None else acc + tk
        x2 = x_ref[16:KC, :]
        jj4 = lax.broadcasted_iota(jnp.int32, (4, 1), 0)
        t2 = jnp.where(x2 > 0, jnp.int32(1) << (jj4 + 16), zero)
        y2 = x2 + pltpu.roll(x2, 3, 0)
        z2 = y2 + pltpu.roll(y2, 2, 0)
        w52 = jnp.where(jj4 == 0, jnp.int32(1) << 24, zero)
        t2 = t2 + jnp.where(z2 > 0, w52, zero)
        out_ref[...] = jnp.sum(acc, axis=0) + jnp.sum(t2, axis=0)

    return pl.pallas_call(
        body,
        grid=(nrows // BT,),
        in_specs=[pl.BlockSpec((KC, BT), lambda i: (0, i))],
        out_specs=pl.BlockSpec((BT,), lambda i: (i,)),
        out_shape=jax.ShapeDtypeStruct((nrows,), jnp.int32),
    )


@functools.lru_cache(maxsize=None)
def _sc_hist_fn(nrows):
    rows_w = nrows // NWORKER
    nchunk = rows_w // CHK
    mesh = plsc.VectorSubcoreMesh(core_axis_name="c", subcore_axis_name="s")

    @functools.partial(
        pl.kernel,
        mesh=mesh,
        compiler_params=pltpu.CompilerParams(
            needs_layout_passes=False, use_tc_tiling_on_sc=False),
        out_type=jax.ShapeDtypeStruct((NWORKER * NHIST,), jnp.float32),
        scratch_types=[
            pltpu.VMEM((CHK,), jnp.int32),
            pltpu.VMEM((CHK,), jnp.int32),
            pltpu.VMEM((NHIST,), jnp.float32),
            pltpu.SemaphoreType.DMA,
            pltpu.SemaphoreType.DMA,
        ],
    )
    def sc_hist(pk_hbm, out_hbm, buf0, buf1, hist, sem0, sem1):
        wid = lax.axis_index("s") * 2 + lax.axis_index("c")
        base = wid * rows_w
        zeros16 = jnp.zeros((NLANE,), jnp.float32)
        for b in range(NGROUP):
            hist[pl.ds(b * NLANE, NLANE)] = zeros16
        lane = lax.iota(jnp.int32, NLANE)
        ones16 = jnp.ones((NLANE,), jnp.float32)

        def compute(buf):
            def body(t, carry):
                p = buf[pl.ds(t * NLANE, NLANE)]
                plsc.addupdate_scatter(
                    hist, [((p & 15) << 4) + lane], ones16)
                for i in range(1, NBLK):
                    plsc.addupdate_scatter(
                        hist,
                        [((p >> (4 * i - 4)) & 240) + (lane + i * 256)],
                        ones16)
                plsc.addupdate_scatter(
                    hist, [((p >> 16) & 496) + (lane + 1280)], ones16)
                return carry

            lax.fori_loop(0, CHK // NLANE, body, 0)

        bufs = (buf0, buf1)
        sems = (sem0, sem1)
        handles = [None] * nchunk
        handles[0] = pltpu.async_copy(
            pk_hbm.at[pl.ds(base, CHK)], bufs[0], sems[0])
        for ci in range(nchunk):
            if ci + 1 < nchunk:
                nb = (ci + 1) & 1
                handles[ci + 1] = pltpu.async_copy(
                    pk_hbm.at[pl.ds(base + (ci + 1) * CHK, CHK)],
                    bufs[nb], sems[nb])
            handles[ci].wait()
            compute(bufs[ci & 1])
        pltpu.sync_copy(hist, out_hbm.at[pl.ds(wid * NHIST, NHIST)])

    return sc_hist


@functools.lru_cache(maxsize=None)
def _tc_entropy_fn(nrows):
    denom = np.float32(nrows + 1e-8)

    def body(p_ref, mt_ref, gt_ref, out_ref):
        x = p_ref[...]
        c = jnp.sum(jnp.sum(x, axis=2), axis=0)
        counts = c.reshape(1, NGROUP)
        sub = jnp.dot(counts, mt_ref[...], preferred_element_type=jnp.float32,
                      precision=jax.lax.Precision.HIGHEST)
        p = sub / denom
        logp = jnp.log(jnp.where(sub > 0, p, 1.0)) * INV_LOG2
        t = jnp.where(sub > 0, -(p * logp), 0.0)
        out_ref[...] = jnp.dot(t, gt_ref[...], preferred_element_type=jnp.float32,
                               precision=jax.lax.Precision.HIGHEST)

    return pl.pallas_call(
        body,
        out_shape=jax.ShapeDtypeStruct((1, 8), jnp.float32),
    )


def kernel(phi):
    nrows = phi.shape[0]
    packed = _tc_pack_fn(nrows)(phi.T)
    partial = _sc_hist_fn(nrows)(packed)
    partial3 = partial.reshape(NWORKER, NGROUP, NLANE)
    res = _tc_entropy_fn(nrows)(partial3, _MT, _GT)
    return res[0, :NBLK], res[0, NBLK]
